# baseline (speedup 1.0000x reference)
import functools

import jax
import jax.numpy as jnp
from jax import lax
from jax.experimental import pallas as pl
from jax.experimental.pallas import tpu as pltpu
from jax.experimental.pallas import tpu_sc as plsc

_EPS = 0.1
_NW = 32
_RPT = 8
_SC_ROWS = _NW * _RPT
_UNROLL = 5


def _row_stats_kernel(x_ref, tgt_ref, loss_ref, margin_ref):
    x = x_ref[...]
    tgt = tgt_ref[...]
    idx = jax.lax.broadcasted_iota(jnp.int32, x.shape, 1)
    eqt = idx == tgt
    out_t = jnp.sum(jnp.where(eqt, x, 0.0), axis=1, keepdims=True)
    m2t = jnp.max(jnp.where(eqt, -jnp.inf, x), axis=1, keepdims=True)
    m1 = jnp.maximum(m2t, out_t)
    lse = m1 + jnp.log(jnp.sum(jnp.exp(x - m1), axis=1, keepdims=True))
    margin = out_t - m2t
    fst = jax.nn.relu(1.0 - margin)
    snd = jax.nn.relu(1.0 - out_t + lse)
    loss_ref[...] = jnp.where(margin >= 0.0, fst, snd)
    margin_ref[...] = margin


def _make_sc_kernel(v):
    nch = 2
    ch = v // nch
    step = 16 * _UNROLL
    mesh = plsc.VectorSubcoreMesh(core_axis_name="c", subcore_axis_name="s")

    @functools.partial(
        pl.kernel,
        out_type=[
            jax.ShapeDtypeStruct((_SC_ROWS * 16,), jnp.float32),
            jax.ShapeDtypeStruct((_SC_ROWS * 16,), jnp.float32),
            jax.ShapeDtypeStruct((_SC_ROWS * 16,), jnp.float32),
        ],
        mesh=mesh,
        scratch_types=[
            pltpu.VMEM((ch,), jnp.float32),
            pltpu.VMEM((ch,), jnp.float32),
            pltpu.VMEM((16,), jnp.int32),
            pltpu.VMEM((_RPT * 16,), jnp.float32),
            pltpu.VMEM((_RPT * 16,), jnp.float32),
            pltpu.VMEM((_RPT * 16,), jnp.float32),
        ],
    )
    def sc_kernel(x_hbm, tgt_hbm, outt_hbm, ml_hbm, se_hbm,
                  buf0, buf1, tgtv, r_outt, r_ml, r_se):
        wid = lax.axis_index("s") * 2 + lax.axis_index("c")
        base = pl.multiple_of(wid * _RPT, 8)
        pltpu.sync_copy(tgt_hbm.at[pl.ds(base, _RPT)],
                        tgtv.at[pl.ds(0, _RPT)])
        lane = lax.broadcasted_iota(jnp.int32, (16,), 0)
        neg_inf = jnp.float32(-jnp.inf)
        tv = tgtv[...]
        for r in range(_RPT):
            t_s = tv[r]
            outt_vec = jnp.zeros((16,), jnp.float32)
            for c, buf in ((0, buf0), (1, buf1)):
                off = pl.multiple_of((base + r) * v + c * ch, 8)
                pltpu.sync_copy(x_hbm.at[pl.ds(off, ch)], buf)
                tl_raw = t_s - c * ch
                sbase = (jnp.clip(tl_raw, 0, ch - 1) // 16) * 16
                chunk16 = buf[pl.ds(sbase, 16)]
                hit = (jnp.broadcast_to(sbase, (16,)) + lane
                       ) == jnp.broadcast_to(tl_raw, (16,))
                outt_vec = jnp.where(hit, chunk16, outt_vec)
                buf[pl.ds(sbase, 16)] = jnp.where(hit, neg_inf, chunk16)

            def body_a(i, acc):
                o = i * step
                for k in range(_UNROLL):
                    acc = jnp.maximum(acc, buf0[pl.ds(o + k * 16, 16)])
                    acc = jnp.maximum(acc, buf1[pl.ds(o + k * 16, 16)])
                return acc

            acc_a = lax.fori_loop(0, ch // step, body_a,
                                  jnp.full((16,), neg_inf, jnp.float32))

            def body_b(i, acc):
                o = i * step
                for k in range(_UNROLL):
                    acc = acc + jnp.exp(buf0[pl.ds(o + k * 16, 16)] - acc_a)
                    acc = acc + jnp.exp(buf1[pl.ds(o + k * 16, 16)] - acc_a)
                return acc

            acc_b = lax.fori_loop(0, ch // step, body_b,
                                  jnp.zeros((16,), jnp.float32))
            r_outt[pl.ds(r * 16, 16)] = outt_vec
            r_ml[pl.ds(r * 16, 16)] = acc_a
            r_se[pl.ds(r * 16, 16)] = acc_b
        obase = pl.multiple_of(base * 16, 8)
        pltpu.sync_copy(r_outt, outt_hbm.at[pl.ds(obase, _RPT * 16)])
        pltpu.sync_copy(r_ml, ml_hbm.at[pl.ds(obase, _RPT * 16)])
        pltpu.sync_copy(r_se, se_hbm.at[pl.ds(obase, _RPT * 16)])

    return sc_kernel


def _finish_kernel(outt_ref, ml_ref, se_ref, ltc_ref, mtc_ref,
                   loss_ref, margin_ref):
    oh = outt_ref[...]
    ml = ml_ref[...]
    se_l = se_ref[...]
    outt = jnp.sum(oh, axis=1, keepdims=True)
    m2t = jnp.max(ml, axis=1, keepdims=True)
    margin_sc = outt - m2t
    m1 = jnp.maximum(m2t, outt)
    se = (jnp.sum(se_l * jnp.exp(ml - m1), axis=1, keepdims=True)
          + jnp.exp(outt - m1))
    lse = m1 + jnp.log(se)
    fst = jax.nn.relu(1.0 - margin_sc)
    snd = jax.nn.relu(1.0 - outt + lse)
    loss_sc = jnp.where(margin_sc >= 0.0, fst, snd)
    loss_ref[...] = jnp.concatenate([loss_sc, ltc_ref[...]], axis=0)
    margin_ref[...] = jnp.concatenate([margin_sc, mtc_ref[...]], axis=0)


def _select_kernel(lc_ref, lr_ref, margin_ref, out_ref):
    b = lc_ref.shape[0]
    lc = lc_ref[...]
    lr = lr_ref[...]
    margin = margin_ref[...]
    neg = jnp.sum((margin < 0.0).astype(jnp.float32))
    threshold = (1.0 - _EPS) ** 2 * b + (1.0 - _EPS) * neg
    ii = jax.lax.broadcasted_iota(jnp.int32, (b, b), 0)
    jj = jax.lax.broadcasted_iota(jnp.int32, (b, b), 1)
    before = ((lr < lc) | ((lr == lc) & (jj <= ii))).astype(jnp.float32)
    rank = jnp.sum(before, axis=1, keepdims=True) - 1.0
    psum = jnp.sum(before * lr, axis=1, keepdims=True)
    sel = (psum <= threshold + 1.0 - rank).astype(jnp.float32)
    npcl1 = jnp.sum(lc * sel)
    cnt = jnp.sum(sel)
    npcl2 = threshold - cnt
    out_ref[...] = jnp.full((1, 1), jnp.maximum(npcl1, npcl2) / cnt,
                            jnp.float32)


def _tc_row_stats(output, tgt2d, row_start, rb):
    b, v = output.shape
    nrows = b - row_start
    blk0 = row_start // rb
    return pl.pallas_call(
        _row_stats_kernel,
        grid=(nrows // rb,),
        in_specs=[
            pl.BlockSpec((rb, v), lambda i: (i + blk0, 0)),
            pl.BlockSpec((rb, 1), lambda i: (i + blk0, 0)),
        ],
        out_specs=[
            pl.BlockSpec((rb, 1), lambda i: (i, 0)),
            pl.BlockSpec((rb, 1), lambda i: (i, 0)),
        ],
        out_shape=[
            jax.ShapeDtypeStruct((nrows, 1), jnp.float32),
            jax.ShapeDtypeStruct((nrows, 1), jnp.float32),
        ],
    )(output, tgt2d)


def _select(loss, margin):
    b = loss.shape[0]
    return pl.pallas_call(
        _select_kernel,
        out_shape=jax.ShapeDtypeStruct((1, 1), jnp.float32),
    )(loss, loss.reshape(1, b), margin)[0, 0]


def kernel(output, target):
    b, v = output.shape
    tgt = target.astype(jnp.int32)
    tgt2d = tgt.reshape(b, 1)
    hybrid = (b % 64 == 0 and b >= 2 * _SC_ROWS
              and v % (2 * 16 * _UNROLL) == 0 and (v // 2) * 4 <= 200000)
    if not hybrid:
        rb = min(64, b)
        loss, margin = _tc_row_stats(output, tgt2d, 0, rb)
        return _select(loss, margin)

    x_flat = output.reshape(b * v)
    outt, ml, se = _make_sc_kernel(v)(x_flat, tgt[:_SC_ROWS])
    loss_tc, margin_tc = _tc_row_stats(output, tgt2d, _SC_ROWS, 64)
    loss, margin = pl.pallas_call(
        _finish_kernel,
        out_shape=[
            jax.ShapeDtypeStruct((b, 1), jnp.float32),
            jax.ShapeDtypeStruct((b, 1), jnp.float32),
        ],
    )(outt.reshape(_SC_ROWS, 16), ml.reshape(_SC_ROWS, 16),
      se.reshape(_SC_ROWS, 16), loss_tc, margin_tc)
    return _select(loss, margin)


# speedup vs baseline: 2.0033x; 2.0033x over previous
import functools

import jax
import jax.numpy as jnp
from jax import lax
from jax.experimental import pallas as pl
from jax.experimental.pallas import tpu as pltpu
from jax.experimental.pallas import tpu_sc as plsc

_EPS = 0.1
_NW = 32
_RPT = 8
_SC_ROWS = _NW * _RPT
_UNROLL = 5


def _row_stats_kernel(x_ref, tgt_ref, loss_ref, margin_ref):
    x = x_ref[...]
    tgt = tgt_ref[...]
    idx = jax.lax.broadcasted_iota(jnp.int32, x.shape, 1)
    eqt = idx == tgt
    out_t = jnp.sum(jnp.where(eqt, x, 0.0), axis=1, keepdims=True)
    m2t = jnp.max(jnp.where(eqt, -jnp.inf, x), axis=1, keepdims=True)
    m1 = jnp.maximum(m2t, out_t)
    lse = m1 + jnp.log(jnp.sum(jnp.exp(x - m1), axis=1, keepdims=True))
    margin = out_t - m2t
    fst = jax.nn.relu(1.0 - margin)
    snd = jax.nn.relu(1.0 - out_t + lse)
    loss_ref[...] = jnp.where(margin >= 0.0, fst, snd)
    margin_ref[...] = margin


def _make_sc_kernel(b, v):
    vt = (v // 128) * 128
    ch = 9088
    nch = vt // ch
    mesh = plsc.VectorSubcoreMesh(core_axis_name="c", subcore_axis_name="s")

    @functools.partial(
        pl.kernel,
        out_type=[
            jax.ShapeDtypeStruct((_SC_ROWS * 16,), jnp.float32),
            jax.ShapeDtypeStruct((_SC_ROWS * nch * 16,), jnp.float32),
            jax.ShapeDtypeStruct((_SC_ROWS * nch * 16,), jnp.float32),
        ],
        mesh=mesh,
        scratch_types=[
            pltpu.VMEM((8, ch), jnp.float32),
            pltpu.VMEM((16,), jnp.int32),
            pltpu.VMEM((_RPT * 16,), jnp.float32),
            pltpu.VMEM((_RPT * nch * 16,), jnp.float32),
            pltpu.VMEM((_RPT * nch * 16,), jnp.float32),
        ],
    )
    def sc_kernel(x_hbm, tgt_hbm, outt_hbm, ml_hbm, se_hbm,
                  buf, tgtv, r_outt, r_ml, r_se):
        wid = lax.axis_index("s") * 2 + lax.axis_index("c")
        base = pl.multiple_of(wid * _RPT, 8)
        pltpu.sync_copy(tgt_hbm.at[pl.ds(base, _RPT)],
                        tgtv.at[pl.ds(0, _RPT)])
        lane = lax.broadcasted_iota(jnp.int32, (16,), 0)
        neg_inf = jnp.float32(-jnp.inf)
        tv = tgtv[...]
        oh = [jnp.zeros((16,), jnp.float32) for _ in range(_RPT)]
        for c in range(nch):
            pltpu.sync_copy(x_hbm.at[pl.ds(base, 8), pl.ds(c * ch, ch)], buf)
            for r in range(_RPT):
                t_s = tv[r]
                tl_raw = t_s - c * ch
                sbase = (jnp.clip(tl_raw, 0, ch - 1) // 16) * 16
                chunk16 = buf[r, pl.ds(sbase, 16)]
                hit = (jnp.broadcast_to(sbase, (16,)) + lane
                       ) == jnp.broadcast_to(tl_raw, (16,))
                oh[r] = jnp.where(hit, chunk16, oh[r])
                buf[r, pl.ds(sbase, 16)] = jnp.where(hit, neg_inf, chunk16)
            for r in range(_RPT):
                def body_a(i, acc, r=r):
                    o = i * 64
                    for k in range(4):
                        acc = jnp.maximum(acc, buf[r, pl.ds(o + k * 16, 16)])
                    return acc

                ml_rc = lax.fori_loop(0, ch // 64, body_a,
                                      jnp.full((16,), neg_inf, jnp.float32))

                def body_b(i, acc, r=r, m=ml_rc):
                    o = i * 64
                    for k in range(4):
                        acc = acc + jnp.exp(buf[r, pl.ds(o + k * 16, 16)] - m)
                    return acc

                se_rc = lax.fori_loop(0, ch // 64, body_b,
                                      jnp.zeros((16,), jnp.float32))
                r_ml[pl.ds((r * nch + c) * 16, 16)] = ml_rc
                r_se[pl.ds((r * nch + c) * 16, 16)] = se_rc
        for r in range(_RPT):
            r_outt[pl.ds(r * 16, 16)] = oh[r]
        pltpu.sync_copy(r_outt,
                        outt_hbm.at[pl.ds(pl.multiple_of(base * 16, 8),
                                          _RPT * 16)])
        pltpu.sync_copy(r_ml,
                        ml_hbm.at[pl.ds(pl.multiple_of(base * nch * 16, 8),
                                        _RPT * nch * 16)])
        pltpu.sync_copy(r_se,
                        se_hbm.at[pl.ds(pl.multiple_of(base * nch * 16, 8),
                                        _RPT * nch * 16)])

    return sc_kernel, vt, nch


def _finish_kernel(vt, oh_ref, ml_ref, se_ref, strip_ref, tgt_ref,
                   ltc_ref, mtc_ref, loss_ref, margin_ref):
    oh = oh_ref[...]
    ml = ml_ref[...]
    se_l = se_ref[...]
    strip = strip_ref[...]
    tgt = tgt_ref[...]
    scol = jax.lax.broadcasted_iota(jnp.int32, strip.shape, 1) + vt
    eqt = scol == tgt
    strip_nt = jnp.where(eqt, -jnp.inf, strip)
    outt = (jnp.sum(oh, axis=1, keepdims=True)
            + jnp.sum(jnp.where(eqt, strip, 0.0), axis=1, keepdims=True))
    m2t = jnp.maximum(jnp.max(ml, axis=1, keepdims=True),
                      jnp.max(strip_nt, axis=1, keepdims=True))
    margin_sc = outt - m2t
    m1 = jnp.maximum(m2t, outt)
    se = (jnp.sum(se_l * jnp.exp(ml - m1), axis=1, keepdims=True)
          + jnp.sum(jnp.exp(strip_nt - m1), axis=1, keepdims=True)
          + jnp.exp(outt - m1))
    lse = m1 + jnp.log(se)
    fst = jax.nn.relu(1.0 - margin_sc)
    snd = jax.nn.relu(1.0 - outt + lse)
    loss_sc = jnp.where(margin_sc >= 0.0, fst, snd)
    loss_ref[...] = jnp.concatenate([loss_sc, ltc_ref[...]], axis=0)
    margin_ref[...] = jnp.concatenate([margin_sc, mtc_ref[...]], axis=0)


def _select_kernel(lc_ref, lr_ref, margin_ref, out_ref):
    b = lc_ref.shape[0]
    lc = lc_ref[...]
    lr = lr_ref[...]
    margin = margin_ref[...]
    neg = jnp.sum((margin < 0.0).astype(jnp.float32))
    threshold = (1.0 - _EPS) ** 2 * b + (1.0 - _EPS) * neg
    ii = jax.lax.broadcasted_iota(jnp.int32, (b, b), 0)
    jj = jax.lax.broadcasted_iota(jnp.int32, (b, b), 1)
    before = ((lr < lc) | ((lr == lc) & (jj <= ii))).astype(jnp.float32)
    rank = jnp.sum(before, axis=1, keepdims=True) - 1.0
    psum = jnp.sum(before * lr, axis=1, keepdims=True)
    sel = (psum <= threshold + 1.0 - rank).astype(jnp.float32)
    npcl1 = jnp.sum(lc * sel)
    cnt = jnp.sum(sel)
    npcl2 = threshold - cnt
    out_ref[...] = jnp.full((1, 1), jnp.maximum(npcl1, npcl2) / cnt,
                            jnp.float32)


def _tc_row_stats(output, tgt2d, row_start, rb):
    b, v = output.shape
    nrows = b - row_start
    blk0 = row_start // rb
    return pl.pallas_call(
        _row_stats_kernel,
        grid=(nrows // rb,),
        in_specs=[
            pl.BlockSpec((rb, v), lambda i: (i + blk0, 0)),
            pl.BlockSpec((rb, 1), lambda i: (i + blk0, 0)),
        ],
        out_specs=[
            pl.BlockSpec((rb, 1), lambda i: (i, 0)),
            pl.BlockSpec((rb, 1), lambda i: (i, 0)),
        ],
        out_shape=[
            jax.ShapeDtypeStruct((nrows, 1), jnp.float32),
            jax.ShapeDtypeStruct((nrows, 1), jnp.float32),
        ],
    )(output, tgt2d)


def _select(loss, margin):
    b = loss.shape[0]
    return pl.pallas_call(
        _select_kernel,
        out_shape=jax.ShapeDtypeStruct((1, 1), jnp.float32),
    )(loss, loss.reshape(1, b), margin)[0, 0]


def kernel(output, target):
    b, v = output.shape
    tgt = target.astype(jnp.int32)
    tgt2d = tgt.reshape(b, 1)
    hybrid = b % 64 == 0 and b >= 2 * _SC_ROWS and v == 100000
    if not hybrid:
        rb = min(64, b)
        loss, margin = _tc_row_stats(output, tgt2d, 0, rb)
        return _select(loss, margin)

    sc_k, vt, nch = _make_sc_kernel(b, v)
    outt, ml, se = sc_k(output, tgt[:_SC_ROWS])
    strip = output[:_SC_ROWS, vt:]
    loss_tc, margin_tc = _tc_row_stats(output, tgt2d, _SC_ROWS, 64)
    loss, margin = pl.pallas_call(
        functools.partial(_finish_kernel, vt),
        out_shape=[
            jax.ShapeDtypeStruct((b, 1), jnp.float32),
            jax.ShapeDtypeStruct((b, 1), jnp.float32),
        ],
    )(outt.reshape(_SC_ROWS, 16), ml.reshape(_SC_ROWS, nch * 16),
      se.reshape(_SC_ROWS, nch * 16), strip, tgt2d[:_SC_ROWS],
      loss_tc, margin_tc)
    return _select(loss, margin)
